# baseline (speedup 1.0000x reference)
import functools

import jax
import jax.numpy as jnp
from jax import lax
from jax.experimental import pallas as pl
from jax.experimental.pallas import tpu as pltpu
from jax.experimental.pallas import tpu_sc as plsc

NC = 2
NS = 16
NW = NC * NS
LANES = 16
CHUNK = 128


def _sc_mesh():
    return plsc.VectorSubcoreMesh(
        core_axis_name="c", subcore_axis_name="s",
        num_cores=NC, num_subcores=NS)


FRAC0 = 0.367


def _make_deg_kernel(n_pad, t):
    n_slab = n_pad // NS

    @functools.partial(
        pl.kernel,
        out_type=jax.ShapeDtypeStruct((NC, n_pad), jnp.float32),
        mesh=_sc_mesh(),
        scratch_types=[
            pltpu.VMEM((t, CHUNK), jnp.int32),
            pltpu.VMEM((n_slab,), jnp.float32),
            pltpu.VMEM_SHARED((n_pad,), jnp.float32),
        ],
    )
    def deg_kernel(dst_hbm, deg_hbm, dst_v, ones_v, acc_sh):
        c = lax.axis_index("c")
        s = lax.axis_index("s")
        wid = c * NS + s
        pltpu.sync_copy(dst_hbm.at[pl.ds(wid * t, t)], dst_v)

        zv = jnp.zeros((LANES,), jnp.float32)

        def fill_zero(r, carry):
            ones_v[pl.ds(r * LANES, LANES)] = zv
            return carry
        lax.fori_loop(0, n_slab // LANES, fill_zero, 0)

        pltpu.sync_copy(ones_v, acc_sh.at[pl.ds(s * n_slab, n_slab)])
        plsc.subcore_barrier()

        ov = jnp.full((LANES,), 1.0, jnp.float32)

        def fill_ones(r, carry):
            ones_v[pl.ds(r * LANES, LANES)] = ov
            return carry
        lax.fori_loop(0, CHUNK // LANES, fill_ones, 0)

        def body(j, carry):
            pltpu.sync_copy(ones_v.at[pl.ds(0, CHUNK)],
                            acc_sh.at[dst_v.at[j]], add=True)
            return carry
        lax.fori_loop(0, t, body, 0)

        plsc.subcore_barrier()
        pltpu.sync_copy(acc_sh.at[pl.ds(s * n_slab, n_slab)],
                        deg_hbm.at[c, pl.ds(s * n_slab, n_slab)])

    return deg_kernel


IB = 8


def _make_scatter_kernel(n_pad, t0, t1, d):
    n_slab = n_pad // NS
    n_zc = n_slab // CHUNK
    tmax = max(t0, t1)

    @functools.partial(
        pl.kernel,
        out_type=jax.ShapeDtypeStruct((NC, n_pad, d), jnp.float32),
        mesh=_sc_mesh(),
        scratch_types=[
            pltpu.VMEM((tmax, CHUNK), jnp.int32),
            pltpu.VMEM((tmax, CHUNK), jnp.int32),
            pltpu.VMEM((CHUNK, d), jnp.float32),
            pltpu.VMEM_SHARED((n_pad, d), jnp.float32),
            pltpu.SemaphoreType.DMA,
        ],
    )
    def scatter_kernel(z_hbm, src_hbm, dst_hbm, p_hbm,
                       src_v, dst_v, rows_v, acc_sh, sem):
        c = lax.axis_index("c")
        s = lax.axis_index("s")
        base = jnp.where(c == 0, s * t0, NS * t0 + s * t1)
        tcount = jnp.where(c == 0, t0, t1)
        pltpu.sync_copy(src_hbm.at[pl.ds(base, tmax)], src_v)
        pltpu.sync_copy(dst_hbm.at[pl.ds(base, tmax)], dst_v)

        zv = jnp.zeros((LANES,), jnp.float32)

        def fill_zero(r, carry):
            for q in range(d // LANES):
                rows_v[r, pl.ds(q * LANES, LANES)] = zv
            return carry
        lax.fori_loop(0, CHUNK, fill_zero, 0)

        for i in range(n_zc):
            pltpu.sync_copy(
                rows_v, acc_sh.at[pl.ds(s * n_slab + i * CHUNK, CHUNK)])
        plsc.subcore_barrier()

        def body(j, carry):
            pltpu.async_copy(z_hbm.at[src_v.at[j]], rows_v, sem).wait()
            pltpu.sync_copy(rows_v, acc_sh.at[dst_v.at[j]], add=True)
            return carry
        lax.fori_loop(0, tcount, body, 0)

        plsc.subcore_barrier()
        pltpu.sync_copy(acc_sh.at[pl.ds(s * n_slab, n_slab)],
                        p_hbm.at[c, pl.ds(s * n_slab, n_slab)])

    return scatter_kernel


def _mm_body(x_ref, w_ref, deg_ref, z_ref):
    bm = x_ref.shape[0]
    d = deg_ref[0, 0] + deg_ref[1, 0] + 1.0
    dinv = lax.rsqrt(jnp.maximum(d, 1e-12)).reshape(bm, 1)
    xw = jnp.dot(x_ref[:], w_ref[:], preferred_element_type=jnp.float32)
    z_ref[:] = xw * dinv


def _fin_body(p_ref, z_ref, deg_ref, b_ref, o_ref):
    bm = z_ref.shape[0]
    d = deg_ref[0, 0] + deg_ref[1, 0] + 1.0
    dinv = lax.rsqrt(jnp.maximum(d, 1e-12)).reshape(bm, 1)
    agg = p_ref[0] + p_ref[1] + z_ref[:]
    o_ref[:] = agg * dinv + b_ref[:]


def kernel(x, edge_index, W, b):
    n, d_in = x.shape
    d_out = W.shape[1]
    e = edge_index.shape[1]

    n_pad = ((n + 1 + 2047) // 2048) * 2048
    e_pad = -(-e // (NS * CHUNK * 8)) * (NS * CHUNK * 8)
    t = e_pad // (NW * CHUNK)
    bm = 256

    src = edge_index[0]
    dst = edge_index[1]
    pad = e_pad - e
    if pad:
        src = jnp.concatenate([src, jnp.full((pad,), n, jnp.int32)])
        spread = n + 1 + jnp.arange(pad, dtype=jnp.int32) % (n_pad - n - 1)
        dst = jnp.concatenate([dst, spread])
    n_chunks = e_pad // CHUNK
    per_pair = n_chunks // NS
    t0 = 8 * int(round(per_pair * FRAC0 / 8))
    t0 = min(max(t0, 8), per_pair - 8)
    t1 = per_pair - t0
    src2 = src.reshape(n_chunks, CHUNK)
    dst2 = dst.reshape(n_chunks, CHUNK)
    x_pad = jnp.concatenate(
        [x, jnp.zeros((n_pad - n, d_in), x.dtype)], axis=0)

    deg = _make_deg_kernel(n_pad, t)(dst2)
    deg3 = deg.reshape(NC, n_pad // bm, 1, bm)

    z = pl.pallas_call(
        _mm_body,
        grid=(n_pad // bm,),
        in_specs=[
            pl.BlockSpec((bm, d_in), lambda i: (i, 0)),
            pl.BlockSpec((d_in, d_out), lambda i: (0, 0)),
            pl.BlockSpec((NC, 1, 1, bm), lambda i: (0, i, 0, 0)),
        ],
        out_specs=pl.BlockSpec((bm, d_out), lambda i: (i, 0)),
        out_shape=jax.ShapeDtypeStruct((n_pad, d_out), jnp.float32),
    )(x_pad, W, deg3)

    p = _make_scatter_kernel(n_pad, t0, t1, d_out)(z, src2, dst2)

    out = pl.pallas_call(
        _fin_body,
        grid=(n_pad // bm,),
        in_specs=[
            pl.BlockSpec((NC, bm, d_out), lambda i: (0, i, 0)),
            pl.BlockSpec((bm, d_out), lambda i: (i, 0)),
            pl.BlockSpec((NC, 1, 1, bm), lambda i: (0, i, 0, 0)),
            pl.BlockSpec((1, d_out), lambda i: (0, 0)),
        ],
        out_specs=pl.BlockSpec((bm, d_out), lambda i: (i, 0)),
        out_shape=jax.ShapeDtypeStruct((n_pad, d_out), jnp.float32),
    )(p, z, deg3, b.reshape(1, d_out))

    return out[:n]


# speedup vs baseline: 1.7544x; 1.7544x over previous
import functools

import jax
import jax.numpy as jnp
from jax import lax
from jax.experimental import pallas as pl
from jax.experimental.pallas import tpu as pltpu
from jax.experimental.pallas import tpu_sc as plsc

NC = 2
NS = 16
NW = NC * NS
LANES = 16
CHUNK = 128


def _sc_mesh():
    return plsc.VectorSubcoreMesh(
        core_axis_name="c", subcore_axis_name="s",
        num_cores=NC, num_subcores=NS)


def _make_deg_kernel(n_pad, t):
    n_slab = n_pad // NS

    @functools.partial(
        pl.kernel,
        out_type=jax.ShapeDtypeStruct((NC, n_pad), jnp.float32),
        mesh=_sc_mesh(),
        scratch_types=[
            pltpu.VMEM((t, CHUNK), jnp.int32),
            pltpu.VMEM((n_slab,), jnp.float32),
            pltpu.VMEM_SHARED((n_pad,), jnp.float32),
        ],
    )
    def deg_kernel(dst_hbm, deg_hbm, dst_v, ones_v, acc_sh):
        c = lax.axis_index("c")
        s = lax.axis_index("s")
        wid = c * NS + s
        pltpu.sync_copy(dst_hbm.at[wid], dst_v)

        zv = jnp.zeros((LANES,), jnp.float32)

        def fill_zero(r, carry):
            ones_v[pl.ds(r * LANES, LANES)] = zv
            return carry
        lax.fori_loop(0, n_slab // LANES, fill_zero, 0)

        pltpu.sync_copy(ones_v, acc_sh.at[pl.ds(s * n_slab, n_slab)])
        plsc.subcore_barrier()

        ov = jnp.full((LANES,), 1.0, jnp.float32)

        def fill_ones(r, carry):
            ones_v[pl.ds(r * LANES, LANES)] = ov
            return carry
        lax.fori_loop(0, CHUNK // LANES, fill_ones, 0)

        def body(j, carry):
            pltpu.sync_copy(ones_v.at[pl.ds(0, CHUNK)],
                            acc_sh.at[dst_v.at[j]], add=True)
            return carry
        lax.fori_loop(0, t, body, 0)

        plsc.subcore_barrier()
        pltpu.sync_copy(acc_sh.at[pl.ds(s * n_slab, n_slab)],
                        deg_hbm.at[c, pl.ds(s * n_slab, n_slab)])

    return deg_kernel


IB = 8


def _make_scatter_kernel(n_pad, t, d):
    n_slab = n_pad // NS
    n_zc = n_slab // CHUNK

    @functools.partial(
        pl.kernel,
        out_type=jax.ShapeDtypeStruct((NC, n_pad, d), jnp.float32),
        mesh=_sc_mesh(),
        scratch_types=[
            pltpu.VMEM((t, CHUNK), jnp.int32),
            pltpu.VMEM((t, CHUNK), jnp.int32),
            pltpu.VMEM((CHUNK, d), jnp.float32),
            pltpu.VMEM_SHARED((n_pad, d), jnp.float32),
            pltpu.SemaphoreType.DMA,
        ],
    )
    def scatter_kernel(z_hbm, src_hbm, dst_hbm, p_hbm,
                       src_v, dst_v, rows_v, acc_sh, sem):
        c = lax.axis_index("c")
        s = lax.axis_index("s")
        wid = c * NS + s
        pltpu.sync_copy(src_hbm.at[wid], src_v)
        pltpu.sync_copy(dst_hbm.at[wid], dst_v)

        zv = jnp.zeros((LANES,), jnp.float32)

        def fill_zero(r, carry):
            for q in range(d // LANES):
                rows_v[r, pl.ds(q * LANES, LANES)] = zv
            return carry
        lax.fori_loop(0, CHUNK, fill_zero, 0)

        for i in range(n_zc):
            pltpu.sync_copy(
                rows_v, acc_sh.at[pl.ds(s * n_slab + i * CHUNK, CHUNK)])
        plsc.subcore_barrier()

        def body(j, carry):
            pltpu.async_copy(z_hbm.at[src_v.at[j]], rows_v, sem).wait()
            pltpu.sync_copy(rows_v, acc_sh.at[dst_v.at[j]], add=True)
            return carry
        lax.fori_loop(0, t, body, 0)

        plsc.subcore_barrier()
        pltpu.sync_copy(acc_sh.at[pl.ds(s * n_slab, n_slab)],
                        p_hbm.at[c, pl.ds(s * n_slab, n_slab)])

    return scatter_kernel


def _mm_body(x_ref, w_ref, deg_ref, z_ref):
    bm = x_ref.shape[0]
    d = deg_ref[0, 0] + deg_ref[1, 0] + 1.0
    dinv = lax.rsqrt(jnp.maximum(d, 1e-12)).reshape(bm, 1)
    xw = jnp.dot(x_ref[:], w_ref[:], preferred_element_type=jnp.float32)
    z_ref[:] = xw * dinv


def _fin_body(p_ref, z_ref, deg_ref, b_ref, o_ref):
    bm = z_ref.shape[0]
    d = deg_ref[0, 0] + deg_ref[1, 0] + 1.0
    dinv = lax.rsqrt(jnp.maximum(d, 1e-12)).reshape(bm, 1)
    agg = p_ref[0] + p_ref[1] + z_ref[:]
    o_ref[:] = agg * dinv + b_ref[:]


def kernel(x, edge_index, W, b):
    n, d_in = x.shape
    d_out = W.shape[1]
    e = edge_index.shape[1]

    n_pad = ((n + 1 + 2047) // 2048) * 2048
    e_pad = -(-e // (NW * CHUNK)) * (NW * CHUNK)
    t = e_pad // (NW * CHUNK)
    bm = 256

    src = edge_index[0]
    dst = edge_index[1]
    pad = e_pad - e
    if pad:
        src = jnp.concatenate([src, jnp.full((pad,), n, jnp.int32)])
        spread = n + 1 + jnp.arange(pad, dtype=jnp.int32) % (n_pad - n - 1)
        dst = jnp.concatenate([dst, spread])
    src2 = src.reshape(NW, t, CHUNK)
    dst2 = dst.reshape(NW, t, CHUNK)
    x_pad = jnp.concatenate(
        [x, jnp.zeros((n_pad - n, d_in), x.dtype)], axis=0)

    deg = _make_deg_kernel(n_pad, t)(dst2)
    deg3 = deg.reshape(NC, n_pad // bm, 1, bm)

    z = pl.pallas_call(
        _mm_body,
        grid=(n_pad // bm,),
        in_specs=[
            pl.BlockSpec((bm, d_in), lambda i: (i, 0)),
            pl.BlockSpec((d_in, d_out), lambda i: (0, 0)),
            pl.BlockSpec((NC, 1, 1, bm), lambda i: (0, i, 0, 0)),
        ],
        out_specs=pl.BlockSpec((bm, d_out), lambda i: (i, 0)),
        out_shape=jax.ShapeDtypeStruct((n_pad, d_out), jnp.float32),
    )(x_pad, W, deg3)

    p = _make_scatter_kernel(n_pad, t, d_out)(z, src2, dst2)

    out = pl.pallas_call(
        _fin_body,
        grid=(n_pad // bm,),
        in_specs=[
            pl.BlockSpec((NC, bm, d_out), lambda i: (0, i, 0)),
            pl.BlockSpec((bm, d_out), lambda i: (i, 0)),
            pl.BlockSpec((NC, 1, 1, bm), lambda i: (0, i, 0, 0)),
            pl.BlockSpec((1, d_out), lambda i: (0, 0)),
        ],
        out_specs=pl.BlockSpec((bm, d_out), lambda i: (i, 0)),
        out_shape=jax.ShapeDtypeStruct((n_pad, d_out), jnp.float32),
    )(p, z, deg3, b.reshape(1, d_out))

    return out[:n]


# speedup vs baseline: 1.8868x; 1.0754x over previous
import functools

import numpy as np

import jax
import jax.numpy as jnp
from jax import lax
from jax.experimental import pallas as pl
from jax.experimental.pallas import tpu as pltpu
from jax.experimental.pallas import tpu_sc as plsc

NC = 2
NS = 16
NW = NC * NS
LANES = 16
CHUNK = 128


def _sc_mesh():
    return plsc.VectorSubcoreMesh(
        core_axis_name="c", subcore_axis_name="s",
        num_cores=NC, num_subcores=NS)


def _make_deg_kernel(n_pad, t):
    n_slab = n_pad // NS

    @functools.partial(
        pl.kernel,
        out_type=jax.ShapeDtypeStruct((NC, n_pad), jnp.float32),
        mesh=_sc_mesh(),
        scratch_types=[
            pltpu.VMEM((t, CHUNK), jnp.int32),
            pltpu.VMEM((n_slab,), jnp.float32),
            pltpu.VMEM_SHARED((n_pad,), jnp.float32),
        ],
    )
    def deg_kernel(ei_hbm, deg_hbm, dst_v, ones_v, acc_sh):
        c = lax.axis_index("c")
        s = lax.axis_index("s")
        wid = c * NS + s
        pltpu.sync_copy(ei_hbm.at[1, wid], dst_v)

        zv = jnp.zeros((LANES,), jnp.float32)

        def fill_zero(r, carry):
            ones_v[pl.ds(r * LANES, LANES)] = zv
            return carry
        lax.fori_loop(0, n_slab // LANES, fill_zero, 0)

        pltpu.sync_copy(ones_v, acc_sh.at[pl.ds(s * n_slab, n_slab)])
        plsc.subcore_barrier()

        ov = jnp.full((LANES,), 1.0, jnp.float32)

        def fill_ones(r, carry):
            ones_v[pl.ds(r * LANES, LANES)] = ov
            return carry
        lax.fori_loop(0, CHUNK // LANES, fill_ones, 0)

        def body(j, carry):
            pltpu.sync_copy(ones_v.at[pl.ds(0, CHUNK)],
                            acc_sh.at[dst_v.at[j]], add=True)
            return carry
        lax.fori_loop(0, t, body, 0)

        plsc.subcore_barrier()
        pltpu.sync_copy(acc_sh.at[pl.ds(s * n_slab, n_slab)],
                        deg_hbm.at[c, pl.ds(s * n_slab, n_slab)])

    return deg_kernel


IB = 8


def _make_scatter_kernel(n_pad, t, d):
    n_slab = n_pad // NS
    n_zc = n_slab // CHUNK

    @functools.partial(
        pl.kernel,
        out_type=jax.ShapeDtypeStruct((NC, n_pad, d), jnp.float32),
        mesh=_sc_mesh(),
        scratch_types=[
            pltpu.VMEM((t, CHUNK), jnp.int32),
            pltpu.VMEM((t, CHUNK), jnp.int32),
            pltpu.VMEM((CHUNK, d), jnp.float32),
            pltpu.VMEM_SHARED((n_pad, d), jnp.float32),
            pltpu.SemaphoreType.DMA,
        ],
    )
    def scatter_kernel(z_hbm, ei_hbm, p_hbm,
                       src_v, dst_v, rows_v, acc_sh, sem):
        c = lax.axis_index("c")
        s = lax.axis_index("s")
        wid = c * NS + s
        pltpu.sync_copy(ei_hbm.at[0, wid], src_v)
        pltpu.sync_copy(ei_hbm.at[1, wid], dst_v)

        zv = jnp.zeros((LANES,), jnp.float32)

        def fill_zero(r, carry):
            for q in range(d // LANES):
                rows_v[r, pl.ds(q * LANES, LANES)] = zv
            return carry
        lax.fori_loop(0, CHUNK, fill_zero, 0)

        for i in range(n_zc):
            pltpu.sync_copy(
                rows_v, acc_sh.at[pl.ds(s * n_slab + i * CHUNK, CHUNK)])
        plsc.subcore_barrier()

        def body(j, carry):
            pltpu.async_copy(z_hbm.at[src_v.at[j]], rows_v, sem).wait()
            pltpu.sync_copy(rows_v, acc_sh.at[dst_v.at[j]], add=True)
            return carry
        lax.fori_loop(0, t, body, 0)

        plsc.subcore_barrier()
        pltpu.sync_copy(acc_sh.at[pl.ds(s * n_slab, n_slab)],
                        p_hbm.at[c, pl.ds(s * n_slab, n_slab)])

    return scatter_kernel


def _mm_body(x_ref, w_ref, deg_ref, z_ref):
    bm = x_ref.shape[0]
    d = deg_ref[0, 0] + deg_ref[1, 0] + 1.0
    dinv = lax.rsqrt(jnp.maximum(d, 1e-12)).reshape(bm, 1)
    xw = jnp.dot(x_ref[:], w_ref[:], preferred_element_type=jnp.float32)
    z_ref[:] = xw * dinv


def _fin_body(p_ref, z_ref, deg_ref, b_ref, o_ref):
    bm = z_ref.shape[0]
    d = deg_ref[0, 0] + deg_ref[1, 0] + 1.0
    dinv = lax.rsqrt(jnp.maximum(d, 1e-12)).reshape(bm, 1)
    agg = p_ref[0] + p_ref[1] + z_ref[:]
    o_ref[:] = agg * dinv + b_ref[:]


def kernel(x, edge_index, W, b):
    n, d_in = x.shape
    d_out = W.shape[1]
    e = edge_index.shape[1]

    n_pad = ((n + 1 + 2047) // 2048) * 2048
    e_pad = -(-e // (NW * CHUNK)) * (NW * CHUNK)
    t = e_pad // (NW * CHUNK)
    bm = 256

    pad = e_pad - e
    ei = edge_index
    if pad:
        dummy = np.empty((2, pad), np.int32)
        dummy[0] = n
        dummy[1] = n + 1 + np.arange(pad, dtype=np.int32) % (n_pad - n - 1)
        ei = jnp.concatenate([ei, jnp.asarray(dummy)], axis=1)
    ei4 = ei.reshape(2, NW, t, CHUNK)
    x_pad = jnp.concatenate(
        [x, jnp.zeros((n_pad - n, d_in), x.dtype)], axis=0)

    deg = _make_deg_kernel(n_pad, t)(ei4)
    deg3 = deg.reshape(NC, n_pad // bm, 1, bm)

    z = pl.pallas_call(
        _mm_body,
        grid=(n_pad // bm,),
        in_specs=[
            pl.BlockSpec((bm, d_in), lambda i: (i, 0)),
            pl.BlockSpec((d_in, d_out), lambda i: (0, 0)),
            pl.BlockSpec((NC, 1, 1, bm), lambda i: (0, i, 0, 0)),
        ],
        out_specs=pl.BlockSpec((bm, d_out), lambda i: (i, 0)),
        out_shape=jax.ShapeDtypeStruct((n_pad, d_out), jnp.float32),
    )(x_pad, W, deg3)

    p = _make_scatter_kernel(n_pad, t, d_out)(z, ei4)

    out = pl.pallas_call(
        _fin_body,
        grid=(n_pad // bm,),
        in_specs=[
            pl.BlockSpec((NC, bm, d_out), lambda i: (0, i, 0)),
            pl.BlockSpec((bm, d_out), lambda i: (i, 0)),
            pl.BlockSpec((NC, 1, 1, bm), lambda i: (0, i, 0, 0)),
            pl.BlockSpec((1, d_out), lambda i: (0, 0)),
        ],
        out_specs=pl.BlockSpec((bm, d_out), lambda i: (i, 0)),
        out_shape=jax.ShapeDtypeStruct((n_pad, d_out), jnp.float32),
    )(p, z, deg3, b.reshape(1, d_out))

    return out[:n]


# speedup vs baseline: 1.8915x; 1.0025x over previous
import functools

import numpy as np

import jax
import jax.numpy as jnp
from jax import lax
from jax.experimental import pallas as pl
from jax.experimental.pallas import tpu as pltpu
from jax.experimental.pallas import tpu_sc as plsc

NC = 2
NS = 16
NW = NC * NS
LANES = 16
CHUNK = 128


def _sc_mesh():
    return plsc.VectorSubcoreMesh(
        core_axis_name="c", subcore_axis_name="s",
        num_cores=NC, num_subcores=NS)


def _make_deg_kernel(n_pad, t):
    n_slab = n_pad // NS

    @functools.partial(
        pl.kernel,
        out_type=jax.ShapeDtypeStruct((NC, n_pad), jnp.float32),
        mesh=_sc_mesh(),
        scratch_types=[
            pltpu.VMEM((t, CHUNK), jnp.int32),
            pltpu.VMEM((n_slab,), jnp.float32),
            pltpu.VMEM_SHARED((n_pad,), jnp.float32),
        ],
    )
    def deg_kernel(ei_hbm, deg_hbm, dst_v, ones_v, acc_sh):
        c = lax.axis_index("c")
        s = lax.axis_index("s")
        wid = c * NS + s
        pltpu.sync_copy(ei_hbm.at[1, wid], dst_v)

        zv = jnp.zeros((LANES,), jnp.float32)

        def fill_zero(r, carry):
            ones_v[pl.ds(r * LANES, LANES)] = zv
            return carry
        lax.fori_loop(0, n_slab // LANES, fill_zero, 0)

        pltpu.sync_copy(ones_v, acc_sh.at[pl.ds(s * n_slab, n_slab)])
        plsc.subcore_barrier()

        ov = jnp.full((LANES,), 1.0, jnp.float32)

        def fill_ones(r, carry):
            ones_v[pl.ds(r * LANES, LANES)] = ov
            return carry
        lax.fori_loop(0, CHUNK // LANES, fill_ones, 0)

        def body(j, carry):
            pltpu.sync_copy(ones_v.at[pl.ds(0, CHUNK)],
                            acc_sh.at[dst_v.at[j]], add=True)
            return carry
        lax.fori_loop(0, t, body, 0)

        plsc.subcore_barrier()
        pltpu.sync_copy(acc_sh.at[pl.ds(s * n_slab, n_slab)],
                        deg_hbm.at[c, pl.ds(s * n_slab, n_slab)])

    return deg_kernel


IB = 8


def _make_scatter_kernel(n_pad, t, d):
    n_slab = n_pad // NS
    n_zc = n_slab // CHUNK

    @functools.partial(
        pl.kernel,
        out_type=jax.ShapeDtypeStruct((NC, n_pad, d), jnp.float32),
        mesh=_sc_mesh(),
        scratch_types=[
            pltpu.VMEM((t, CHUNK), jnp.int32),
            pltpu.VMEM((t, CHUNK), jnp.int32),
            pltpu.VMEM((CHUNK, d), jnp.float32),
            pltpu.VMEM_SHARED((n_pad, d), jnp.float32),
            pltpu.SemaphoreType.DMA,
        ],
    )
    def scatter_kernel(z_hbm, ei_hbm, p_hbm,
                       src_v, dst_v, rows_v, acc_sh, sem):
        c = lax.axis_index("c")
        s = lax.axis_index("s")
        wid = c * NS + s
        pltpu.sync_copy(ei_hbm.at[0, wid], src_v)
        pltpu.sync_copy(ei_hbm.at[1, wid], dst_v)

        @pl.when(c == 0)
        def _():
            pltpu.sync_copy(z_hbm.at[pl.ds(s * n_slab, n_slab)],
                            acc_sh.at[pl.ds(s * n_slab, n_slab)])

        @pl.when(c == 1)
        def _():
            zv = jnp.zeros((LANES,), jnp.float32)

            def fill_zero(r, carry):
                for q in range(d // LANES):
                    rows_v[r, pl.ds(q * LANES, LANES)] = zv
                return carry
            lax.fori_loop(0, CHUNK, fill_zero, 0)
            for i in range(n_zc):
                pltpu.sync_copy(
                    rows_v, acc_sh.at[pl.ds(s * n_slab + i * CHUNK, CHUNK)])

        plsc.subcore_barrier()

        def body(j, carry):
            pltpu.async_copy(z_hbm.at[src_v.at[j]], rows_v, sem).wait()
            pltpu.sync_copy(rows_v, acc_sh.at[dst_v.at[j]], add=True)
            return carry
        lax.fori_loop(0, t, body, 0)

        plsc.subcore_barrier()
        pltpu.sync_copy(acc_sh.at[pl.ds(s * n_slab, n_slab)],
                        p_hbm.at[c, pl.ds(s * n_slab, n_slab)])

    return scatter_kernel


def _mm_body(x_ref, w_ref, deg_ref, z_ref):
    bm = x_ref.shape[0]
    d = deg_ref[0, 0] + deg_ref[1, 0] + 1.0
    dinv = lax.rsqrt(jnp.maximum(d, 1e-12)).reshape(bm, 1)
    xw = jnp.dot(x_ref[:], w_ref[:], preferred_element_type=jnp.float32)
    z_ref[:] = xw * dinv


def _fin_body(p_ref, deg_ref, b_ref, o_ref):
    bm = p_ref.shape[1]
    d = deg_ref[0, 0] + deg_ref[1, 0] + 1.0
    dinv = lax.rsqrt(jnp.maximum(d, 1e-12)).reshape(bm, 1)
    agg = p_ref[0] + p_ref[1]
    o_ref[:] = agg * dinv + b_ref[:]


def kernel(x, edge_index, W, b):
    n, d_in = x.shape
    d_out = W.shape[1]
    e = edge_index.shape[1]

    n_pad = ((n + 1 + 2047) // 2048) * 2048
    e_pad = -(-e // (NW * CHUNK)) * (NW * CHUNK)
    t = e_pad // (NW * CHUNK)
    bm = 256

    pad = e_pad - e
    ei = edge_index
    if pad:
        dummy = np.empty((2, pad), np.int32)
        dummy[0] = n
        dummy[1] = n + 1 + np.arange(pad, dtype=np.int32) % (n_pad - n - 1)
        ei = jnp.concatenate([ei, jnp.asarray(dummy)], axis=1)
    ei4 = ei.reshape(2, NW, t, CHUNK)
    x_pad = jnp.concatenate(
        [x, jnp.zeros((n_pad - n, d_in), x.dtype)], axis=0)

    deg = _make_deg_kernel(n_pad, t)(ei4)
    deg3 = deg.reshape(NC, n_pad // bm, 1, bm)

    z = pl.pallas_call(
        _mm_body,
        grid=(n_pad // bm,),
        in_specs=[
            pl.BlockSpec((bm, d_in), lambda i: (i, 0)),
            pl.BlockSpec((d_in, d_out), lambda i: (0, 0)),
            pl.BlockSpec((NC, 1, 1, bm), lambda i: (0, i, 0, 0)),
        ],
        out_specs=pl.BlockSpec((bm, d_out), lambda i: (i, 0)),
        out_shape=jax.ShapeDtypeStruct((n_pad, d_out), jnp.float32),
    )(x_pad, W, deg3)

    p = _make_scatter_kernel(n_pad, t, d_out)(z, ei4)

    out = pl.pallas_call(
        _fin_body,
        grid=(n_pad // bm,),
        in_specs=[
            pl.BlockSpec((NC, bm, d_out), lambda i: (0, i, 0)),
            pl.BlockSpec((NC, 1, 1, bm), lambda i: (0, i, 0, 0)),
            pl.BlockSpec((1, d_out), lambda i: (0, 0)),
        ],
        out_specs=pl.BlockSpec((bm, d_out), lambda i: (i, 0)),
        out_shape=jax.ShapeDtypeStruct((n_pad, d_out), jnp.float32),
    )(p, deg3, b.reshape(1, d_out))

    return out[:n]
